# baseline (speedup 1.0000x reference)
import jax
import jax.numpy as jnp
from jax import lax
from jax.experimental import pallas as pl
from jax.experimental.pallas import tpu as pltpu
from jax.experimental.pallas import tpu_sc as plsc

N = 10000
D = 128
E = 320000

NC = 2
NS = 16
NW = NC * NS

CHUNK = 80
EDGES_PER_TILE = E // NW
CHUNKS_PER_TILE = EDGES_PER_TILE // CHUNK

NPAD = 10240
HALF = NPAD // 2
HPAD = 5136
ROWS_PER_TILE = HALF // NS

ZROWS = 40

ROW_BLK = 1024


def _layer_norm(x, g, b, eps=1e-5):
    mu = jnp.mean(x, axis=-1, keepdims=True)
    var = jnp.mean((x - mu) ** 2, axis=-1, keepdims=True)
    return (x - mu) / jnp.sqrt(var + eps) * g + b


def _gelu(x):
    return 0.5 * x * (1.0 + lax.erf(x * (2.0 ** -0.5)))


def _prologue_body(nf_ref, g_ref, b_ref, h_ref):
    h_ref[...] = _gelu(_layer_norm(nf_ref[...], g_ref[...], b_ref[...]))


def _prologue(node_feat, g, b):
    grid = pl.cdiv(N, ROW_BLK)
    return pl.pallas_call(
        _prologue_body,
        grid=(grid,),
        in_specs=[
            pl.BlockSpec((ROW_BLK, D), lambda i: (i, 0)),
            pl.BlockSpec((1, D), lambda i: (0, 0)),
            pl.BlockSpec((1, D), lambda i: (0, 0)),
        ],
        out_specs=pl.BlockSpec((ROW_BLK, D), lambda i: (i, 0)),
        out_shape=jax.ShapeDtypeStruct((N, D), jnp.float32),
    )(node_feat, g.reshape(1, D), b.reshape(1, D))


def _make_sc_body(half_k):
    base = half_k * HALF

    def _sc_body(h_hbm, src_hbm, dst_hbm, sums_out,
                 src_v, dst_v, rows_a, rows_b, zrow_v,
                 sem_a, sem_b, sem_sa, sem_sb, shared_sums):
        c = lax.axis_index("c")
        s = lax.axis_index("s")
        wid = s * NC + c

        def zfill(i, _):
            for j in range(D // 16):
                zrow_v[i, pl.ds(j * 16, 16)] = jnp.zeros((16,), jnp.float32)
            return 0
        lax.fori_loop(0, ZROWS, zfill, 0)


        r0 = s * ROWS_PER_TILE

        def zcopy(q, _):
            pltpu.sync_copy(zrow_v, shared_sums.at[pl.ds(r0 + q * ZROWS, ZROWS)])
            return 0
        lax.fori_loop(0, ROWS_PER_TILE // ZROWS, zcopy, 0)

        @pl.when(s == 0)
        def _():
            pltpu.sync_copy(zrow_v.at[pl.ds(0, 16)],
                            shared_sums.at[pl.ds(HALF, 16)])

        pltpu.sync_copy(src_hbm.at[wid], src_v)
        pltpu.sync_copy(dst_hbm.at[wid], dst_v)

        trash = HALF + s

        def rebase(r, _):
            for kk in range(CHUNK // 16):
                d = dst_v[r, pl.ds(kk * 16, 16)]
                d2 = d - base
                inr = (d2 >= 0) & (d2 < HALF)
                dst_v[r, pl.ds(kk * 16, 16)] = jnp.where(inr, d2, trash)
            return 0
        lax.fori_loop(0, CHUNKS_PER_TILE, rebase, 0)

        plsc.subcore_barrier()

        def drain_a():
            pltpu.make_async_copy(h_hbm.at[pl.ds(0, CHUNK)], rows_a, sem_a).wait()

        def drain_b():
            pltpu.make_async_copy(h_hbm.at[pl.ds(0, CHUNK)], rows_b, sem_b).wait()

        def drain_sa():
            pltpu.make_async_copy(h_hbm.at[pl.ds(0, CHUNK)], rows_a, sem_sa).wait()

        def drain_sb():
            pltpu.make_async_copy(h_hbm.at[pl.ds(0, CHUNK)], rows_b, sem_sb).wait()

        pltpu.async_copy(h_hbm.at[src_v.at[0]], rows_a, sem_a)

        def pipe2(i, _):
            j0 = 2 * i
            drain_a()

            @pl.when(i > 0)
            def _():
                drain_sb()

            pltpu.async_copy(h_hbm.at[src_v.at[j0 + 1]], rows_b, sem_b)
            pltpu.async_copy(rows_a, shared_sums.at[dst_v.at[j0]], sem_sa, add=True)
            drain_b()
            drain_sa()

            @pl.when(j0 + 2 < CHUNKS_PER_TILE)
            def _():
                pltpu.async_copy(h_hbm.at[src_v.at[j0 + 2]], rows_a, sem_a)

            pltpu.async_copy(rows_b, shared_sums.at[dst_v.at[j0 + 1]], sem_sb, add=True)
            return 0
        lax.fori_loop(0, (CHUNKS_PER_TILE - 1) // 2, pipe2, 0)

        j_last = CHUNKS_PER_TILE - 1
        drain_a()
        drain_sb()
        pltpu.sync_copy(rows_a, shared_sums.at[dst_v.at[j_last]], add=True)

        plsc.subcore_barrier()

        pltpu.sync_copy(shared_sums.at[pl.ds(r0, ROWS_PER_TILE)],
                        sums_out.at[c, pl.ds(r0, ROWS_PER_TILE)])

    return _sc_body


def _sc_half_call(half_k):
    mesh = plsc.VectorSubcoreMesh(core_axis_name="c", subcore_axis_name="s")
    return pl.kernel(
        _make_sc_body(half_k),
        out_type=jax.ShapeDtypeStruct((NC, HALF, D), jnp.float32),
        mesh=mesh,
        scratch_types=[
            pltpu.VMEM((CHUNKS_PER_TILE, CHUNK), jnp.int32),
            pltpu.VMEM((CHUNKS_PER_TILE, CHUNK), jnp.int32),
            pltpu.VMEM((CHUNK, D), jnp.float32),
            pltpu.VMEM((CHUNK, D), jnp.float32),
            pltpu.VMEM((ZROWS, D), jnp.float32),
            pltpu.SemaphoreType.DMA,
            pltpu.SemaphoreType.DMA,
            pltpu.SemaphoreType.DMA,
            pltpu.SemaphoreType.DMA,
            pltpu.VMEM_SHARED((HPAD, D), jnp.float32),
        ],
        compiler_params=pltpu.CompilerParams(needs_layout_passes=False),
    )


def _sc_counts_body(dst_hbm, cnt_out, dst_v, cnt_v):
    c = lax.axis_index("c")
    s = lax.axis_index("s")
    wid = s * NC + c

    pltpu.sync_copy(dst_hbm.at[wid], dst_v)

    def czero(i, _):
        cnt_v[pl.ds(i * 16, 16)] = jnp.zeros((16,), jnp.float32)
        return 0
    lax.fori_loop(0, NPAD // 16, czero, 0)

    ones16 = jnp.ones((16,), jnp.float32)

    def crow(j, _):
        for k in range(CHUNK // 16):
            idx16 = dst_v[j, pl.ds(k * 16, 16)]
            plsc.addupdate_scatter(cnt_v, [idx16], ones16)
        return 0
    lax.fori_loop(0, CHUNKS_PER_TILE, crow, 0)

    pltpu.sync_copy(cnt_v, cnt_out.at[wid])


def _sc_counts_call():
    mesh = plsc.VectorSubcoreMesh(core_axis_name="c", subcore_axis_name="s")
    return pl.kernel(
        _sc_counts_body,
        out_type=jax.ShapeDtypeStruct((NW, NPAD), jnp.float32),
        mesh=mesh,
        scratch_types=[
            pltpu.VMEM((CHUNKS_PER_TILE, CHUNK), jnp.int32),
            pltpu.VMEM((NPAD,), jnp.float32),
        ],
        compiler_params=pltpu.CompilerParams(needs_layout_passes=False),
    )


@jax.jit
def _sc_segment(h, src3d, dst3d):
    cnt_p = _sc_counts_call()(dst3d)
    sums_parts = []
    for half_k in range(2):
        sp = _sc_half_call(half_k)(h, src3d, dst3d)
        sums_parts.append(sp)
    sums_p = jnp.concatenate(sums_parts, axis=1)
    return sums_p, cnt_p


def _epilogue_body(nf_ref, h_ref, sp_ref, cp_ref, wl_ref, bl_ref, wr_ref,
                   lng_ref, lnb_ref, w1_ref, b1_ref, w2_ref, b2_ref, out_ref):
    nf = nf_ref[...]
    h = h_ref[...]
    sums = sp_ref[0] + sp_ref[1]
    ones_mat = jnp.ones((NW, D), jnp.float32)
    dn0 = (((0,), (0,)), ((), ()))
    cnt = lax.dot_general(cp_ref[...], ones_mat, dn0,
                          preferred_element_type=jnp.float32)
    mean = sums / jnp.maximum(cnt, 1.0)
    dn = (((1,), (1,)), ((), ()))
    conv = (lax.dot_general(mean, wl_ref[...], dn, preferred_element_type=jnp.float32)
            + bl_ref[...]
            + lax.dot_general(h, wr_ref[...], dn, preferred_element_type=jnp.float32))
    cf = nf + conv
    f = _layer_norm(cf, lng_ref[...], lnb_ref[...])
    f = _gelu(lax.dot_general(f, w1_ref[...], dn, preferred_element_type=jnp.float32)
              + b1_ref[...])
    f = _gelu(lax.dot_general(f, w2_ref[...], dn, preferred_element_type=jnp.float32)
              + b2_ref[...])
    out_ref[...] = nf + f


def _epilogue(node_feat, h, sums_p, cnt_p, W_l, b_l, W_r, ln_g, ln_b, W1, b1, W2, b2):
    grid = pl.cdiv(N, ROW_BLK)
    full = lambda shape: pl.BlockSpec(shape, lambda i: tuple(0 for _ in shape))
    return pl.pallas_call(
        _epilogue_body,
        grid=(grid,),
        in_specs=[
            pl.BlockSpec((ROW_BLK, D), lambda i: (i, 0)),
            pl.BlockSpec((ROW_BLK, D), lambda i: (i, 0)),
            pl.BlockSpec((NC, ROW_BLK, D), lambda i: (0, i, 0)),
            pl.BlockSpec((NW, ROW_BLK), lambda i: (0, i)),
            full((D, D)),
            full((1, D)),
            full((D, D)),
            full((1, D)),
            full((1, D)),
            full((4 * D, D)),
            full((1, 4 * D)),
            full((D, 4 * D)),
            full((1, D)),
        ],
        out_specs=pl.BlockSpec((ROW_BLK, D), lambda i: (i, 0)),
        out_shape=jax.ShapeDtypeStruct((N, D), jnp.float32),
    )(node_feat, h, sums_p, cnt_p, W_l, b_l.reshape(1, D), W_r,
      ln_g.reshape(1, D), ln_b.reshape(1, D), W1, b1.reshape(1, 4 * D),
      W2, b2.reshape(1, D))


def kernel(node_feat, edge_index, layer_idx, norm1_g, norm1_b, W_l, b_l, W_r,
           ln_g, ln_b, W1, b1, W2, b2):
    h = _prologue(node_feat, norm1_g, norm1_b)
    src3d = edge_index[0].reshape(NW, CHUNKS_PER_TILE, CHUNK)
    dst3d = edge_index[1].reshape(NW, CHUNKS_PER_TILE, CHUNK)
    sums_p, cnt_p = _sc_segment(h, src3d, dst3d)
    return _epilogue(node_feat, h, sums_p, cnt_p, W_l, b_l, W_r,
                     ln_g, ln_b, W1, b1, W2, b2)


# speedup vs baseline: 1.0011x; 1.0011x over previous
import jax
import jax.numpy as jnp
from jax import lax
from jax.experimental import pallas as pl
from jax.experimental.pallas import tpu as pltpu
from jax.experimental.pallas import tpu_sc as plsc

N = 10000
D = 128
E = 320000

NC = 2
NS = 16
NW = NC * NS

CHUNK = 80
EDGES_PER_TILE = E // NW
CHUNKS_PER_TILE = EDGES_PER_TILE // CHUNK

NPAD = 10240
HALF = NPAD // 2
HPAD = 5136
ROWS_PER_TILE = HALF // NS

ZROWS = 40

ROW_BLK = 1024


def _layer_norm(x, g, b, eps=1e-5):
    mu = jnp.mean(x, axis=-1, keepdims=True)
    var = jnp.mean((x - mu) ** 2, axis=-1, keepdims=True)
    return (x - mu) / jnp.sqrt(var + eps) * g + b


def _gelu(x):
    return 0.5 * x * (1.0 + lax.erf(x * (2.0 ** -0.5)))


def _prologue_body(nf_ref, g_ref, b_ref, h_ref):
    h_ref[...] = _gelu(_layer_norm(nf_ref[...], g_ref[...], b_ref[...]))


def _prologue(node_feat, g, b):
    grid = pl.cdiv(N, ROW_BLK)
    return pl.pallas_call(
        _prologue_body,
        grid=(grid,),
        in_specs=[
            pl.BlockSpec((ROW_BLK, D), lambda i: (i, 0)),
            pl.BlockSpec((1, D), lambda i: (0, 0)),
            pl.BlockSpec((1, D), lambda i: (0, 0)),
        ],
        out_specs=pl.BlockSpec((ROW_BLK, D), lambda i: (i, 0)),
        out_shape=jax.ShapeDtypeStruct((N, D), jnp.float32),
    )(node_feat, g.reshape(1, D), b.reshape(1, D))


def _make_sc_body(half_k):
    base = half_k * HALF

    def _sc_body(h_hbm, src_hbm, dst_hbm, sums_out,
                 src_v, dst_v, rows_a, rows_b, zrow_v,
                 sem_a, sem_b, shared_sums):
        c = lax.axis_index("c")
        s = lax.axis_index("s")
        wid = s * NC + c

        def zfill(i, _):
            for j in range(D // 16):
                zrow_v[i, pl.ds(j * 16, 16)] = jnp.zeros((16,), jnp.float32)
            return 0
        lax.fori_loop(0, ZROWS, zfill, 0)


        r0 = s * ROWS_PER_TILE

        def zcopy(q, _):
            pltpu.sync_copy(zrow_v, shared_sums.at[pl.ds(r0 + q * ZROWS, ZROWS)])
            return 0
        lax.fori_loop(0, ROWS_PER_TILE // ZROWS, zcopy, 0)

        @pl.when(s == 0)
        def _():
            pltpu.sync_copy(zrow_v.at[pl.ds(0, 16)],
                            shared_sums.at[pl.ds(HALF, 16)])

        pltpu.sync_copy(src_hbm.at[wid], src_v)
        pltpu.sync_copy(dst_hbm.at[wid], dst_v)

        trash = HALF + s

        def rebase(r, _):
            for kk in range(CHUNK // 16):
                d = dst_v[r, pl.ds(kk * 16, 16)]
                d2 = d - base
                inr = (d2 >= 0) & (d2 < HALF)
                dst_v[r, pl.ds(kk * 16, 16)] = jnp.where(inr, d2, trash)
            return 0
        lax.fori_loop(0, CHUNKS_PER_TILE, rebase, 0)

        plsc.subcore_barrier()

        def drain_a():
            pltpu.make_async_copy(h_hbm.at[pl.ds(0, CHUNK)], rows_a, sem_a).wait()

        def drain_b():
            pltpu.make_async_copy(h_hbm.at[pl.ds(0, CHUNK)], rows_b, sem_b).wait()

        pltpu.async_copy(h_hbm.at[src_v.at[0]], rows_a, sem_a)

        def pipe2(i, _):
            j0 = 2 * i
            drain_a()
            pltpu.async_copy(h_hbm.at[src_v.at[j0 + 1]], rows_b, sem_b)
            pltpu.sync_copy(rows_a, shared_sums.at[dst_v.at[j0]], add=True)
            drain_b()

            @pl.when(j0 + 2 < CHUNKS_PER_TILE)
            def _():
                pltpu.async_copy(h_hbm.at[src_v.at[j0 + 2]], rows_a, sem_a)

            pltpu.sync_copy(rows_b, shared_sums.at[dst_v.at[j0 + 1]], add=True)
            return 0
        lax.fori_loop(0, (CHUNKS_PER_TILE - 1) // 2, pipe2, 0)

        j_last = CHUNKS_PER_TILE - 1
        drain_a()
        pltpu.sync_copy(rows_a, shared_sums.at[dst_v.at[j_last]], add=True)

        plsc.subcore_barrier()

        pltpu.sync_copy(shared_sums.at[pl.ds(r0, ROWS_PER_TILE)],
                        sums_out.at[c, pl.ds(r0, ROWS_PER_TILE)])

    return _sc_body


def _sc_half_call(half_k):
    mesh = plsc.VectorSubcoreMesh(core_axis_name="c", subcore_axis_name="s")
    return pl.kernel(
        _make_sc_body(half_k),
        out_type=jax.ShapeDtypeStruct((NC, HALF, D), jnp.float32),
        mesh=mesh,
        scratch_types=[
            pltpu.VMEM((CHUNKS_PER_TILE, CHUNK), jnp.int32),
            pltpu.VMEM((CHUNKS_PER_TILE, CHUNK), jnp.int32),
            pltpu.VMEM((CHUNK, D), jnp.float32),
            pltpu.VMEM((CHUNK, D), jnp.float32),
            pltpu.VMEM((ZROWS, D), jnp.float32),
            pltpu.SemaphoreType.DMA,
            pltpu.SemaphoreType.DMA,
            pltpu.VMEM_SHARED((HPAD, D), jnp.float32),
        ],
        compiler_params=pltpu.CompilerParams(needs_layout_passes=False),
    )


def _sc_counts_body(dst_hbm, cnt_out, dst_v, cnt_v):
    c = lax.axis_index("c")
    s = lax.axis_index("s")
    wid = s * NC + c

    pltpu.sync_copy(dst_hbm.at[wid], dst_v)

    def czero(i, _):
        cnt_v[pl.ds(i * 16, 16)] = jnp.zeros((16,), jnp.float32)
        return 0
    lax.fori_loop(0, NPAD // 16, czero, 0)

    ones16 = jnp.ones((16,), jnp.float32)

    def crow(j, _):
        for k in range(CHUNK // 16):
            idx16 = dst_v[j, pl.ds(k * 16, 16)]
            plsc.addupdate_scatter(cnt_v, [idx16], ones16)
        return 0
    lax.fori_loop(0, CHUNKS_PER_TILE, crow, 0)

    pltpu.sync_copy(cnt_v, cnt_out.at[wid])


def _sc_counts_call():
    mesh = plsc.VectorSubcoreMesh(core_axis_name="c", subcore_axis_name="s")
    return pl.kernel(
        _sc_counts_body,
        out_type=jax.ShapeDtypeStruct((NW, NPAD), jnp.float32),
        mesh=mesh,
        scratch_types=[
            pltpu.VMEM((CHUNKS_PER_TILE, CHUNK), jnp.int32),
            pltpu.VMEM((NPAD,), jnp.float32),
        ],
        compiler_params=pltpu.CompilerParams(needs_layout_passes=False),
    )


@jax.jit
def _sc_segment(h, src3d, dst3d):
    cnt_p = _sc_counts_call()(dst3d)
    sums_parts = []
    for half_k in range(2):
        sp = _sc_half_call(half_k)(h, src3d, dst3d)
        sums_parts.append(sp)
    sums_p = jnp.concatenate(sums_parts, axis=1)
    return sums_p, cnt_p


def _epilogue_body(nf_ref, h_ref, sp_ref, cp_ref, wl_ref, bl_ref, wr_ref,
                   lng_ref, lnb_ref, w1_ref, b1_ref, w2_ref, b2_ref, out_ref):
    nf = nf_ref[...]
    h = h_ref[...]
    sums = sp_ref[0] + sp_ref[1]
    ones_mat = jnp.ones((NW, D), jnp.float32)
    dn0 = (((0,), (0,)), ((), ()))
    cnt = lax.dot_general(cp_ref[...], ones_mat, dn0,
                          preferred_element_type=jnp.float32)
    mean = sums / jnp.maximum(cnt, 1.0)
    dn = (((1,), (1,)), ((), ()))
    conv = (lax.dot_general(mean, wl_ref[...], dn, preferred_element_type=jnp.float32)
            + bl_ref[...]
            + lax.dot_general(h, wr_ref[...], dn, preferred_element_type=jnp.float32))
    cf = nf + conv
    f = _layer_norm(cf, lng_ref[...], lnb_ref[...])
    f = _gelu(lax.dot_general(f, w1_ref[...], dn, preferred_element_type=jnp.float32)
              + b1_ref[...])
    f = _gelu(lax.dot_general(f, w2_ref[...], dn, preferred_element_type=jnp.float32)
              + b2_ref[...])
    out_ref[...] = nf + f


def _epilogue(node_feat, h, sums_p, cnt_p, W_l, b_l, W_r, ln_g, ln_b, W1, b1, W2, b2):
    grid = pl.cdiv(N, ROW_BLK)
    full = lambda shape: pl.BlockSpec(shape, lambda i: tuple(0 for _ in shape))
    return pl.pallas_call(
        _epilogue_body,
        grid=(grid,),
        in_specs=[
            pl.BlockSpec((ROW_BLK, D), lambda i: (i, 0)),
            pl.BlockSpec((ROW_BLK, D), lambda i: (i, 0)),
            pl.BlockSpec((NC, ROW_BLK, D), lambda i: (0, i, 0)),
            pl.BlockSpec((NW, ROW_BLK), lambda i: (0, i)),
            full((D, D)),
            full((1, D)),
            full((D, D)),
            full((1, D)),
            full((1, D)),
            full((4 * D, D)),
            full((1, 4 * D)),
            full((D, 4 * D)),
            full((1, D)),
        ],
        out_specs=pl.BlockSpec((ROW_BLK, D), lambda i: (i, 0)),
        out_shape=jax.ShapeDtypeStruct((N, D), jnp.float32),
    )(node_feat, h, sums_p, cnt_p, W_l, b_l.reshape(1, D), W_r,
      ln_g.reshape(1, D), ln_b.reshape(1, D), W1, b1.reshape(1, 4 * D),
      W2, b2.reshape(1, D))


def kernel(node_feat, edge_index, layer_idx, norm1_g, norm1_b, W_l, b_l, W_r,
           ln_g, ln_b, W1, b1, W2, b2):
    h = _prologue(node_feat, norm1_g, norm1_b)
    src3d = edge_index[0].reshape(NW, CHUNKS_PER_TILE, CHUNK)
    dst3d = edge_index[1].reshape(NW, CHUNKS_PER_TILE, CHUNK)
    sums_p, cnt_p = _sc_segment(h, src3d, dst3d)
    return _epilogue(node_feat, h, sums_p, cnt_p, W_l, b_l, W_r,
                     ln_g, ln_b, W1, b1, W2, b2)
